# baseline (speedup 1.0000x reference)
import functools

import jax
import jax.numpy as jnp
from jax import lax
from jax.experimental import pallas as pl
from jax.experimental.pallas import tpu as pltpu
from jax.experimental.pallas import tpu_sc as plsc

V_BLK = 4096


def _make_sc_gather(V, D, B):
    info = plsc.get_sparse_core_info()
    NC, NS = info.num_cores, info.num_subcores
    NW = NC * NS
    b_per_w = B // NW
    mesh = plsc.VectorSubcoreMesh(core_axis_name="c", subcore_axis_name="s")

    @functools.partial(
        pl.kernel,
        mesh=mesh,
        out_type=jax.ShapeDtypeStruct((B, D), jnp.bfloat16),
        scratch_types=[
            pltpu.VMEM((b_per_w,), jnp.int32),
            pltpu.VMEM((b_per_w, D), jnp.bfloat16),
            pltpu.SemaphoreType.DMA,
        ],
        compiler_params=pltpu.CompilerParams(use_tc_tiling_on_sc=False),
    )
    def gather_kernel(idx_hbm, table_hbm, out_hbm, idx_v, rows_v, sem):
        wid = lax.axis_index("s") * NC + lax.axis_index("c")
        base = wid * b_per_w
        pltpu.sync_copy(idx_hbm.at[pl.ds(base, b_per_w)], idx_v)
        pltpu.async_copy(table_hbm.at[idx_v], rows_v, sem).wait()
        pltpu.sync_copy(rows_v, out_hbm.at[pl.ds(base, b_per_w)])

    return gather_kernel


def _proj_kernel(wt_ref, embt_ref, b_ref, out_ref):
    out_ref[...] = lax.dot_general(
        wt_ref[...], embt_ref[...].astype(jnp.float32),
        (((0,), (0,)), ((), ())),
        preferred_element_type=jnp.float32,
    ) + jnp.transpose(b_ref[...])


@jax.jit
def kernel(x, emb_table, fc_w, fc_b):
    V, D = emb_table.shape
    B = x.shape[0]
    idx = x.astype(jnp.int32)

    embedded = _make_sc_gather(V, D, B)(idx, emb_table.astype(jnp.bfloat16))

    nv = pl.cdiv(V, V_BLK)
    out_t = pl.pallas_call(
        _proj_kernel,
        grid=(nv,),
        in_specs=[
            pl.BlockSpec((D, V_BLK), lambda v: (0, v)),
            pl.BlockSpec((D, B), lambda v: (0, 0)),
            pl.BlockSpec((1, V_BLK), lambda v: (0, v)),
        ],
        out_specs=pl.BlockSpec((V_BLK, B), lambda v: (v, 0)),
        out_shape=jax.ShapeDtypeStruct((V, B), jnp.float32),
        compiler_params=pltpu.CompilerParams(
            dimension_semantics=("arbitrary",),
        ),
    )(fc_w.T, embedded.T, fc_b.reshape(1, V))
    return out_t.T


# speedup vs baseline: 1.1445x; 1.1445x over previous
import functools

import jax
import jax.numpy as jnp
from jax import lax
from jax.experimental import pallas as pl
from jax.experimental.pallas import tpu as pltpu
from jax.experimental.pallas import tpu_sc as plsc

V_BLK = 5120


def _make_sc_gather(V, D, B):
    info = plsc.get_sparse_core_info()
    NC, NS = info.num_cores, info.num_subcores
    NW = NC * NS
    b_per_w = B // NW
    mesh = plsc.VectorSubcoreMesh(core_axis_name="c", subcore_axis_name="s")

    @functools.partial(
        pl.kernel,
        mesh=mesh,
        out_type=jax.ShapeDtypeStruct((B, D), jnp.float32),
        scratch_types=[
            pltpu.VMEM((b_per_w,), jnp.int32),
            pltpu.VMEM((b_per_w, D), jnp.float32),
            pltpu.SemaphoreType.DMA,
        ],
        compiler_params=pltpu.CompilerParams(use_tc_tiling_on_sc=False),
    )
    def gather_kernel(idx_hbm, table_hbm, out_hbm, idx_v, rows_v, sem):
        wid = lax.axis_index("s") * NC + lax.axis_index("c")
        base = wid * b_per_w
        pltpu.sync_copy(idx_hbm.at[pl.ds(base, b_per_w)], idx_v)
        pltpu.async_copy(table_hbm.at[idx_v], rows_v, sem).wait()
        pltpu.sync_copy(rows_v, out_hbm.at[pl.ds(base, b_per_w)])

    return gather_kernel


def _proj_kernel(wt_ref, embt_ref, b_ref, out_ref):
    out_ref[...] = lax.dot_general(
        wt_ref[...], embt_ref[...], (((0,), (0,)), ((), ())),
        preferred_element_type=jnp.float32,
    ) + jnp.transpose(b_ref[...])


@jax.jit
def kernel(x, emb_table, fc_w, fc_b):
    V, D = emb_table.shape
    B = x.shape[0]
    idx = x.astype(jnp.int32)

    embedded = _make_sc_gather(V, D, B)(idx, emb_table)

    nv = pl.cdiv(V, V_BLK)
    out_t = pl.pallas_call(
        _proj_kernel,
        grid=(nv,),
        in_specs=[
            pl.BlockSpec((D, V_BLK), lambda v: (0, v)),
            pl.BlockSpec((D, B), lambda v: (0, 0)),
            pl.BlockSpec((1, V_BLK), lambda v: (0, v)),
        ],
        out_specs=pl.BlockSpec((V_BLK, B), lambda v: (v, 0)),
        out_shape=jax.ShapeDtypeStruct((V, B), jnp.float32),
        compiler_params=pltpu.CompilerParams(
            dimension_semantics=("arbitrary",),
        ),
    )(fc_w.T, embedded.T, fc_b.reshape(1, V))
    return out_t.T


# speedup vs baseline: 1.1881x; 1.0381x over previous
import functools

import jax
import jax.numpy as jnp
from jax import lax
from jax.experimental import pallas as pl
from jax.experimental.pallas import tpu as pltpu
from jax.experimental.pallas import tpu_sc as plsc

V_BLK = 5120


def _make_sc_gather(V, DP, B):
    info = plsc.get_sparse_core_info()
    NC, NS = info.num_cores, info.num_subcores
    NW = NC * NS
    b_per_w = B // NW
    mesh = plsc.VectorSubcoreMesh(core_axis_name="c", subcore_axis_name="s")

    @functools.partial(
        pl.kernel,
        mesh=mesh,
        out_type=jax.ShapeDtypeStruct((B, DP), jnp.float32),
        scratch_types=[
            pltpu.VMEM((b_per_w,), jnp.int32),
            pltpu.VMEM((b_per_w, DP), jnp.float32),
            pltpu.SemaphoreType.DMA,
        ],
        compiler_params=pltpu.CompilerParams(use_tc_tiling_on_sc=True),
    )
    def gather_kernel(idx_hbm, table_hbm, out_hbm, idx_v, rows_v, sem):
        wid = lax.axis_index("s") * NC + lax.axis_index("c")
        base = wid * b_per_w
        pltpu.sync_copy(idx_hbm.at[pl.ds(base, b_per_w)], idx_v)
        pltpu.async_copy(table_hbm.at[idx_v], rows_v, sem).wait()
        pltpu.sync_copy(rows_v, out_hbm.at[pl.ds(base, b_per_w)])

    return gather_kernel


def _proj_kernel(wt_ref, embt_ref, b_ref, out_ref):
    out_ref[...] = lax.dot_general(
        wt_ref[...], embt_ref[...], (((0,), (0,)), ((), ())),
        preferred_element_type=jnp.float32,
    ) + jnp.transpose(b_ref[...])


@jax.jit
def kernel(x, emb_table, fc_w, fc_b):
    V, D = emb_table.shape
    B = x.shape[0]
    idx = x.astype(jnp.int32)

    table_pad = jnp.pad(emb_table, ((0, 0), (0, 128 - D)))
    emb_pad = _make_sc_gather(V, 128, B)(idx, table_pad)
    embt_full = emb_pad.T

    nv = pl.cdiv(V, V_BLK)
    out_t = pl.pallas_call(
        _proj_kernel,
        grid=(nv,),
        in_specs=[
            pl.BlockSpec((D, V_BLK), lambda v: (0, v)),
            pl.BlockSpec((D, B), lambda v: (0, 0)),
            pl.BlockSpec((1, V_BLK), lambda v: (0, v)),
        ],
        out_specs=pl.BlockSpec((V_BLK, B), lambda v: (v, 0)),
        out_shape=jax.ShapeDtypeStruct((V, B), jnp.float32),
        compiler_params=pltpu.CompilerParams(
            dimension_semantics=("arbitrary",),
        ),
    )(fc_w.T, embt_full, fc_b.reshape(1, V))
    return out_t.T
